# Initial kernel scaffold; baseline (speedup 1.0000x reference)
import jax
import jax.numpy as jnp
from jax.experimental import pallas as pl


def kernel(input_embs, pos_table, ln_weight, ln_bias):
    raise NotImplementedError("write your pallas kernel here")


# speedup vs baseline: 3.5495x; 3.5495x over previous
import functools

import jax
import jax.numpy as jnp
from jax.experimental import pallas as pl

_EPS = 1e-5
_ROWS = 512


def _ln_body(x_ref, pos_ref, w_ref, b_ref, o_ref):
    x = x_ref[0] + pos_ref[...]
    mean = jnp.mean(x, axis=-1, keepdims=True)
    xc = x - mean
    var = jnp.mean(xc * xc, axis=-1, keepdims=True)
    normed = xc * jax.lax.rsqrt(var + _EPS)
    o_ref[0] = normed * w_ref[...] + b_ref[...]


@jax.jit
def kernel(input_embs, pos_table, ln_weight, ln_bias):
    B, S, H = input_embs.shape
    rows = _ROWS if S % _ROWS == 0 else S
    grid = (S // rows, B)
    return pl.pallas_call(
        _ln_body,
        grid=grid,
        in_specs=[
            pl.BlockSpec((1, rows, H), lambda i, b: (b, i, 0)),
            pl.BlockSpec((rows, H), lambda i, b: (i, 0)),
            pl.BlockSpec((1, H), lambda i, b: (0, 0)),
            pl.BlockSpec((1, H), lambda i, b: (0, 0)),
        ],
        out_specs=pl.BlockSpec((1, rows, H), lambda i, b: (b, i, 0)),
        out_shape=jax.ShapeDtypeStruct((B, S, H), input_embs.dtype),
    )(
        input_embs,
        pos_table[:S],
        ln_weight.reshape(1, H),
        ln_bias.reshape(1, H),
    )


# speedup vs baseline: 4.0115x; 1.1302x over previous
import functools

import jax
import jax.numpy as jnp
from jax.experimental import pallas as pl

_EPS = 1e-5
_ROWS = 1024


def _ln_body(x_ref, pos_ref, w_ref, b_ref, o_ref):
    x = x_ref[0] + pos_ref[...]
    mean = jnp.mean(x, axis=-1, keepdims=True)
    xc = x - mean
    var = jnp.mean(xc * xc, axis=-1, keepdims=True)
    normed = xc * jax.lax.rsqrt(var + _EPS)
    o_ref[0] = normed * w_ref[...] + b_ref[...]


@jax.jit
def kernel(input_embs, pos_table, ln_weight, ln_bias):
    B, S, H = input_embs.shape
    rows = _ROWS if S % _ROWS == 0 else S
    grid = (S // rows, B)
    return pl.pallas_call(
        _ln_body,
        grid=grid,
        in_specs=[
            pl.BlockSpec((1, rows, H), lambda i, b: (b, i, 0)),
            pl.BlockSpec((rows, H), lambda i, b: (i, 0)),
            pl.BlockSpec((1, H), lambda i, b: (0, 0)),
            pl.BlockSpec((1, H), lambda i, b: (0, 0)),
        ],
        out_specs=pl.BlockSpec((1, rows, H), lambda i, b: (b, i, 0)),
        out_shape=jax.ShapeDtypeStruct((B, S, H), input_embs.dtype),
    )(
        input_embs,
        pos_table[:S],
        ln_weight.reshape(1, H),
        ln_bias.reshape(1, H),
    )


# speedup vs baseline: 4.1695x; 1.0394x over previous
import functools

import jax
import jax.numpy as jnp
from jax.experimental import pallas as pl

_EPS = 1e-5
_ROWS = 2048


def _ln_body(x_ref, pos_ref, w_ref, b_ref, o_ref):
    x = x_ref[0] + pos_ref[...]
    mean = jnp.mean(x, axis=-1, keepdims=True)
    xc = x - mean
    var = jnp.mean(xc * xc, axis=-1, keepdims=True)
    normed = xc * jax.lax.rsqrt(var + _EPS)
    o_ref[0] = normed * w_ref[...] + b_ref[...]


@jax.jit
def kernel(input_embs, pos_table, ln_weight, ln_bias):
    B, S, H = input_embs.shape
    rows = _ROWS if S % _ROWS == 0 else S
    grid = (S // rows, B)
    return pl.pallas_call(
        _ln_body,
        grid=grid,
        in_specs=[
            pl.BlockSpec((1, rows, H), lambda i, b: (b, i, 0)),
            pl.BlockSpec((rows, H), lambda i, b: (i, 0)),
            pl.BlockSpec((1, H), lambda i, b: (0, 0)),
            pl.BlockSpec((1, H), lambda i, b: (0, 0)),
        ],
        out_specs=pl.BlockSpec((1, rows, H), lambda i, b: (b, i, 0)),
        out_shape=jax.ShapeDtypeStruct((B, S, H), input_embs.dtype),
    )(
        input_embs,
        pos_table[:S],
        ln_weight.reshape(1, H),
        ln_bias.reshape(1, H),
    )
